# baseline (speedup 1.0000x reference)
import functools

import jax
import jax.numpy as jnp
from jax import lax
from jax.experimental import pallas as pl
from jax.experimental.pallas import tpu as pltpu
import jax.experimental.pallas.tpu_sc as plsc

V = 1000000
D = 64
B = 16384
C = 20
NEG = 5

NC = 2
NS = 16
NW = NC * NS
BPW = B // NW
CH = 128
NCHUNK = BPW // CH
LANES = 16
DV = D // LANES


def _dot_chunk(a_ref, b_ref, dst_ref, dst_base, scale):
    def g_body(g, _):
        rows = jnp.arange(LANES, dtype=jnp.int32) + g * LANES

        def d_body(d, tot):
            cols = jnp.zeros((LANES,), jnp.int32) + d
            a = plsc.load_gather(a_ref, [rows, cols])
            b = plsc.load_gather(b_ref, [rows, cols])
            return tot + a * b

        tot = lax.fori_loop(0, D, d_body, jnp.zeros((LANES,), jnp.float32),
                            unroll=8)
        dst_ref[pl.ds(dst_base + g * LANES, LANES)] = tot * scale
        return 0

    lax.fori_loop(0, CH // LANES, g_body, 0)


def _sc_body(ctx_hbm, tgt_hbm, neg_hbm, win_hbm, wout_hbm, pos_out, neg_out,
             idx_v, acc_v, tmp_v, pos_v, negsc_v, sem):
    wid = lax.axis_index("s") * NC + lax.axis_index("c")
    base = wid * BPW
    inv_c = jnp.float32(1.0 / C)

    for ch in range(NCHUNK):
        off = base + ch * CH

        pltpu.sync_copy(ctx_hbm.at[pl.ds(off, CH)], idx_v)
        pltpu.async_copy(win_hbm.at[idx_v], acc_v, sem).wait()
        for c in range(1, C):
            pltpu.sync_copy(ctx_hbm.at[pl.ds(c * B + off, CH)], idx_v)
            pltpu.async_copy(win_hbm.at[idx_v], tmp_v, sem).wait()

            def acc_body(r, _):
                for j in range(DV):
                    sl = pl.ds(j * LANES, LANES)
                    acc_v[r, sl] = acc_v[r, sl] + tmp_v[r, sl]
                return 0

            lax.fori_loop(0, CH, acc_body, 0, unroll=2)

        pltpu.sync_copy(tgt_hbm.at[pl.ds(off, CH)], idx_v)
        pltpu.async_copy(wout_hbm.at[idx_v], tmp_v, sem).wait()
        _dot_chunk(acc_v, tmp_v, pos_v, ch * CH, inv_c)

        for n in range(NEG):
            pltpu.sync_copy(neg_hbm.at[pl.ds(n * B + off, CH)], idx_v)
            pltpu.async_copy(wout_hbm.at[idx_v], tmp_v, sem).wait()
            _dot_chunk(acc_v, tmp_v, negsc_v, n * BPW + ch * CH, inv_c)

    pltpu.sync_copy(pos_v, pos_out.at[pl.ds(base, BPW)])
    for n in range(NEG):
        pltpu.sync_copy(negsc_v.at[pl.ds(n * BPW, BPW)],
                        neg_out.at[pl.ds(n * B + base, BPW)])


_sc_scores = functools.partial(
    pl.kernel,
    out_type=(
        jax.ShapeDtypeStruct((B,), jnp.float32),
        jax.ShapeDtypeStruct((NEG * B,), jnp.float32),
    ),
    mesh=plsc.VectorSubcoreMesh(
        core_axis_name="c", subcore_axis_name="s", num_cores=NC, num_subcores=NS
    ),
    compiler_params=pltpu.CompilerParams(
        needs_layout_passes=False, use_tc_tiling_on_sc=False
    ),
    scratch_types=[
        pltpu.VMEM((CH,), jnp.int32),
        pltpu.VMEM((CH, D), jnp.float32),
        pltpu.VMEM((CH, D), jnp.float32),
        pltpu.VMEM((BPW,), jnp.float32),
        pltpu.VMEM((NEG * BPW,), jnp.float32),
        pltpu.SemaphoreType.DMA,
    ],
)(_sc_body)


def _loss_body(pos_ref, neg_ref, pl_ref, nl_ref):
    x = pos_ref[...]
    ls_p = jnp.where(x < 0.0, x, 0.0) - jnp.log1p(jnp.exp(-jnp.abs(x)))
    pl_ref[...] = (-jnp.sum(ls_p) * jnp.float32(1.0 / B)).reshape(1, 1)
    y = -neg_ref[...]
    ls_n = jnp.where(y < 0.0, y, 0.0) - jnp.log1p(jnp.exp(-jnp.abs(y)))
    nl_ref[...] = (-jnp.sum(ls_n) * jnp.float32(1.0 / (B * NEG))).reshape(1, 1)


_loss_call = pl.pallas_call(
    _loss_body,
    out_shape=(
        jax.ShapeDtypeStruct((1, 1), jnp.float32),
        jax.ShapeDtypeStruct((1, 1), jnp.float32),
    ),
)


def kernel(context_words, target_words, negative_words, W_in, W_out):
    ctx_flat = context_words.astype(jnp.int32).T.reshape(C * B)
    neg_flat = negative_words.astype(jnp.int32).T.reshape(NEG * B)
    tgt = target_words.astype(jnp.int32)
    pos_sc, neg_sc = _sc_scores(ctx_flat, tgt, neg_flat, W_in, W_out)
    pos_loss, neg_loss = _loss_call(
        pos_sc.reshape(128, 128), neg_sc.reshape(NEG * 128, 128)
    )
    return (pos_loss[0, 0], neg_loss[0, 0])
